# Initial kernel scaffold; baseline (speedup 1.0000x reference)
import functools

import jax
import jax.numpy as jnp
from jax import lax
from jax.experimental import pallas as pl
from jax.experimental.pallas import tpu as pltpu
from jax.experimental.pallas import tpu_sc as plsc

N_ROWS = 20000
D = 512
K = 16
L = 16
NC = 2
NS = 16
NW = NC * NS
RPW = N_ROWS // NW
B = 125
NBLK = RPW // B
NCH = D // L

NEG = jnp.float32(-3.0e38)

_mesh = plsc.VectorSubcoreMesh(core_axis_name="c", subcore_axis_name="s")


@functools.partial(
    pl.kernel,
    out_type=(
        jax.ShapeDtypeStruct((N_ROWS, K), jnp.float32),
        jax.ShapeDtypeStruct((N_ROWS, K), jnp.int32),
    ),
    mesh=_mesh,
    scratch_types=[
        pltpu.VMEM((B, D), jnp.float32),
        pltpu.VMEM((B, K), jnp.float32),
        pltpu.VMEM((B, K), jnp.int32),
    ],
)
def _softmax_topk(x_hbm, probs_hbm, inds_hbm, x_v, p_v, i_v):
    wid = lax.axis_index("s") * NC + lax.axis_index("c")
    base = wid * RPW
    lane = lax.iota(jnp.int32, L)

    def do_block(b, carry_b):
        row0 = base + b * B
        pltpu.sync_copy(x_hbm.at[pl.ds(row0, B)], x_v)

        def do_row(r, carry_r):
            top_v = jnp.full((L,), NEG, jnp.float32)
            top_i = jnp.zeros((L,), jnp.int32)
            for c in range(NCH):
                v = x_v[r, pl.ds(c * L, L)]
                sv, si = plsc.sort_key_val(v, lane + c * L, descending=True)
                rv = lax.rev(top_v, (0,))
                ri = lax.rev(top_i, (0,))
                m = sv >= rv
                mv = jnp.where(m, sv, rv)
                mi = jnp.where(m, si, ri)
                top_v, top_i = plsc.sort_key_val(mv, mi, descending=True)
            mx = jnp.max(top_v)
            acc = jnp.zeros((L,), jnp.float32)
            for c in range(NCH):
                acc = acc + jnp.exp(x_v[r, pl.ds(c * L, L)] - mx)
            s = jnp.sum(acc)
            p_v[r] = jnp.exp(top_v - mx) / s
            i_v[r] = top_i
            return carry_r

        lax.fori_loop(0, B, do_row, 0)
        pltpu.sync_copy(p_v, probs_hbm.at[pl.ds(row0, B)])
        pltpu.sync_copy(i_v, inds_hbm.at[pl.ds(row0, B)])
        return carry_b

    lax.fori_loop(0, NBLK, do_block, 0)


def kernel(x, boxes, features):
    probs, inds = _softmax_topk(x)
    return probs, inds, boxes, features


# speedup vs baseline: 9.0952x; 9.0952x over previous
import functools

import jax
import jax.numpy as jnp
from jax import lax
from jax.experimental import pallas as pl
from jax.experimental.pallas import tpu as pltpu
from jax.experimental.pallas import tpu_sc as plsc

N_ROWS = 20000
D = 512
K = 16
L = 16
NC = 2
NS = 16
NW = NC * NS
B = 40
NB = N_ROWS // B
NCH = D // L

NEG = -3.0e38

_mesh = plsc.VectorSubcoreMesh(core_axis_name="c", subcore_axis_name="s")


@functools.partial(
    pl.kernel,
    out_type=(
        jax.ShapeDtypeStruct((N_ROWS, K), jnp.float32),
        jax.ShapeDtypeStruct((N_ROWS, K), jnp.int32),
    ),
    mesh=_mesh,
    compiler_params=pltpu.CompilerParams(needs_layout_passes=False),
    scratch_types=[
        pltpu.VMEM((B, D), jnp.float32),
        pltpu.VMEM((B, K), jnp.float32),
        pltpu.VMEM((B, K), jnp.int32),
    ],
)
def _softmax_topk(x_hbm, probs_hbm, inds_hbm, x_v, p_v, i_v):
    wid = lax.axis_index("s") * NC + lax.axis_index("c")
    nblk = (NB - wid + NW - 1) // NW
    lane = lax.iota(jnp.int32, L)

    def do_block(k, carry_b):
        row0 = (wid + k * NW) * B
        pltpu.sync_copy(x_hbm.at[pl.ds(row0, B)], x_v)

        def do_row(r, carry_r):
            top_v = jnp.full((L,), NEG, jnp.float32)
            top_i = jnp.zeros((L,), jnp.int32)
            for c in range(NCH):
                v = x_v[r, pl.ds(c * L, L)]
                sv, si = plsc.sort_key_val(v, lane + c * L, descending=True)
                rv = lax.rev(top_v, (0,))
                ri = lax.rev(top_i, (0,))
                m = sv >= rv
                mv = jnp.where(m, sv, rv)
                mi = jnp.where(m, si, ri)
                top_v, top_i = plsc.sort_key_val(mv, mi, descending=True)
            mx = jnp.max(top_v)
            acc = jnp.zeros((L,), jnp.float32)
            for c in range(NCH):
                acc = acc + jnp.exp(x_v[r, pl.ds(c * L, L)] - mx)
            s = jnp.sum(acc)
            p_v[r] = jnp.exp(top_v - mx) / s
            i_v[r] = top_i
            return carry_r

        lax.fori_loop(0, B, do_row, 0)
        pltpu.sync_copy(p_v, probs_hbm.at[pl.ds(row0, B)])
        pltpu.sync_copy(i_v, inds_hbm.at[pl.ds(row0, B)])
        return carry_b

    lax.fori_loop(0, nblk, do_block, 0)


def kernel(x, boxes, features):
    probs, inds = _softmax_topk(x)
    return probs, inds, boxes, features


# speedup vs baseline: 9.7330x; 1.0701x over previous
import functools

import jax
import jax.numpy as jnp
from jax import lax
from jax.experimental import pallas as pl
from jax.experimental.pallas import tpu as pltpu
from jax.experimental.pallas import tpu_sc as plsc

N_ROWS = 20000
D = 512
K = 16
L = 16
NC = 2
NS = 16
NW = NC * NS
B = 40
NB = N_ROWS // B
NCH = D // L
U = 4

NEG = -3.0e38

_mesh = plsc.VectorSubcoreMesh(core_axis_name="c", subcore_axis_name="s")


@functools.partial(
    pl.kernel,
    out_type=(
        jax.ShapeDtypeStruct((N_ROWS, K), jnp.float32),
        jax.ShapeDtypeStruct((N_ROWS, K), jnp.int32),
    ),
    mesh=_mesh,
    compiler_params=pltpu.CompilerParams(needs_layout_passes=False),
    scratch_types=[
        pltpu.VMEM((B, D), jnp.float32),
        pltpu.VMEM((B, K), jnp.float32),
        pltpu.VMEM((B, K), jnp.int32),
    ],
)
def _softmax_topk(x_hbm, probs_hbm, inds_hbm, x_v, p_v, i_v):
    wid = lax.axis_index("s") * NC + lax.axis_index("c")
    nblk = (NB - wid + NW - 1) // NW
    lane = lax.iota(jnp.int32, L)

    def do_block(k, carry_b):
        row0 = (wid + k * NW) * B
        pltpu.sync_copy(x_hbm.at[pl.ds(row0, B)], x_v)

        def do_rows(rr, carry_r):
            r0 = rr * U
            tops_v = [jnp.full((L,), NEG, jnp.float32) for _ in range(U)]
            tops_i = [jnp.zeros((L,), jnp.int32) for _ in range(U)]
            for c in range(NCH):
                for u in range(U):
                    v = x_v[r0 + u, pl.ds(c * L, L)]
                    sv, si = plsc.sort_key_val(v, lane + c * L, descending=True)
                    m = sv >= tops_v[u]
                    mv = jnp.where(m, sv, tops_v[u])
                    mi = jnp.where(m, si, tops_i[u])
                    tops_v[u], tops_i[u] = plsc.sort_key_val(mv, mi)
            mxs = [jnp.max(tops_v[u]) for u in range(U)]
            accs = [jnp.zeros((L,), jnp.float32) for _ in range(U)]
            for c in range(NCH):
                for u in range(U):
                    accs[u] = accs[u] + jnp.exp(x_v[r0 + u, pl.ds(c * L, L)] - mxs[u])
            for u in range(U):
                s = jnp.sum(accs[u])
                p_v[r0 + u] = lax.rev(jnp.exp(tops_v[u] - mxs[u]) / s, (0,))
                i_v[r0 + u] = lax.rev(tops_i[u], (0,))
            return carry_r

        lax.fori_loop(0, B // U, do_rows, 0)
        pltpu.sync_copy(p_v, probs_hbm.at[pl.ds(row0, B)])
        pltpu.sync_copy(i_v, inds_hbm.at[pl.ds(row0, B)])
        return carry_b

    lax.fori_loop(0, nblk, do_block, 0)


def kernel(x, boxes, features):
    probs, inds = _softmax_topk(x)
    return probs, inds, boxes, features
